# baseline (speedup 1.0000x reference)
import functools
import math

import jax
import jax.numpy as jnp
from jax import lax
from jax.experimental import pallas as pl
from jax.experimental.pallas import tpu as pltpu
from jax.experimental.pallas import tpu_sc as plsc

N = 10000
E = 160000
D_IN = 256
D_OUT = 256
H = 8
DH = 32
SCALE = 1.0 / math.sqrt(DH)

NP = 10016
BLK = 128
EC = 20480
NBLK = EC // BLK
E_PAD = 8 * EC
ROWW = 40
ACC_ROWS = 2 * NP
RPT = ACC_ROWS // 16


def _proj_body(x_ref, wq_ref, wk_ref, wv_ref, q_ref, kv_ref):
    xb = x_ref[...]
    q = jnp.dot(xb, wq_ref[...], preferred_element_type=jnp.float32)
    k = jnp.dot(xb, wk_ref[...], preferred_element_type=jnp.float32)
    v = jnp.dot(xb, wv_ref[...], preferred_element_type=jnp.float32)
    for h in range(H):
        q_ref[h] = q[:, h * DH:(h + 1) * DH]
        kv_ref[h, :, 0:DH] = k[:, h * DH:(h + 1) * DH]
        kv_ref[h, :, DH:2 * DH] = v[:, h * DH:(h + 1) * DH]


def _project(x, wq, wk, wv):
    nb = 2000
    grid = (N // nb,)
    return pl.pallas_call(
        _proj_body,
        grid=grid,
        in_specs=[
            pl.BlockSpec((nb, D_IN), lambda i: (i, 0)),
            pl.BlockSpec((D_IN, D_OUT), lambda i: (0, 0)),
            pl.BlockSpec((D_IN, D_OUT), lambda i: (0, 0)),
            pl.BlockSpec((D_IN, D_OUT), lambda i: (0, 0)),
        ],
        out_specs=[
            pl.BlockSpec((H, nb, DH), lambda i: (0, i, 0)),
            pl.BlockSpec((H, nb, 2 * DH), lambda i: (0, i, 0)),
        ],
        out_shape=[jax.ShapeDtypeStruct((H, N, DH), jnp.float32),
                   jax.ShapeDtypeStruct((H, N, 2 * DH), jnp.float32)],
    )(x, wq, wk, wv)


def _edge_body(q_hbm, kv_hbm, ei_hbm, out_hbm,
               idxb, sidx, qb, kvb, msg, zb, acc,
               ix0, ix1, ix2, ix3, g0, g1, g2, g3, s0, s1):
    ix = (ix0, ix1, ix2, ix3)
    g = (g0, g1, g2, g3)
    sg = (s0, s1)
    c = lax.axis_index("c")
    s = lax.axis_index("s")
    hloc = s // 8
    chunk = s % 8

    zeros16 = jnp.zeros((16,), jnp.float32)

    def _zrow(r, carry):
        for off in (0, 16, 24):
            zb[r, pl.ds(off, 16)] = zeros16
        return carry
    lax.fori_loop(0, BLK, _zrow, 0)

    base = s * RPT
    ebase = chunk * EC
    aoff = hloc * NP
    lanes = lax.iota(jnp.int32, 16)
    rot_idx = {r: (lanes + r) & 15 for r in (8, 4, 2, 1)}

    def _rot(a, r):
        return a.at[rot_idx[r]].get(mode="promise_in_bounds")

    def _pass(p, carry):
        head = c * 4 + p * 2 + hloc
        qoff = head * N

        nfull = RPT // BLK
        rem = RPT - nfull * BLK

        def _zacc(i, carry2):
            pltpu.async_copy(zb, acc.at[pl.ds(base + i * BLK, BLK)], ix[0])
            return carry2
        lax.fori_loop(0, nfull, _zacc, 0)
        if rem:
            pltpu.async_copy(zb.at[pl.ds(0, rem)],
                             acc.at[pl.ds(base + nfull * BLK, rem)], ix[0])

        def _zwait(i, carry2):
            pltpu.make_async_copy(
                zb, acc.at[pl.ds(base + i * BLK, BLK)], ix[0]).wait()
            return carry2
        lax.fori_loop(0, nfull, _zwait, 0)
        if rem:
            pltpu.make_async_copy(
                zb.at[pl.ds(0, rem)],
                acc.at[pl.ds(base + nfull * BLK, rem)], ix[0]).wait()
        plsc.subcore_barrier()

        def idx_fire(blk_i, slot):
            e0 = ebase + blk_i * BLK
            pltpu.async_copy(ei_hbm.at[0, pl.ds(e0, BLK)], idxb.at[slot, 0],
                             ix[slot])
            pltpu.async_copy(ei_hbm.at[1, pl.ds(e0, BLK)], idxb.at[slot, 1],
                             ix[slot])

        def idx_wait(blk_i, slot):
            e0 = ebase + blk_i * BLK
            pltpu.make_async_copy(ei_hbm.at[0, pl.ds(e0, BLK)],
                                  idxb.at[slot, 0], ix[slot]).wait()
            pltpu.make_async_copy(ei_hbm.at[1, pl.ds(e0, BLK)],
                                  idxb.at[slot, 1], ix[slot]).wait()

        def offsets_and_gather(slot):
            nmax = jnp.full((16,), N - 1, jnp.int32)
            for t in range(BLK // 16):
                sl = pl.ds(t * 16, 16)
                idxb[slot, 2, sl] = jnp.minimum(idxb[slot, 1, sl], nmax) + qoff
                idxb[slot, 3, sl] = jnp.minimum(idxb[slot, 0, sl], nmax) + qoff
            pltpu.async_copy(q_hbm.at[idxb.at[slot, 2]], qb.at[slot], g[slot])
            pltpu.async_copy(kv_hbm.at[idxb.at[slot, 3]], kvb.at[slot],
                             g[slot])

        def gather_wait(slot):
            pltpu.make_async_copy(q_hbm.at[idxb.at[slot, 2]], qb.at[slot],
                                  g[slot]).wait()
            pltpu.make_async_copy(kv_hbm.at[idxb.at[slot, 3]], kvb.at[slot],
                                  g[slot]).wait()

        def scatter_wait(pp):
            pltpu.make_async_copy(msg.at[pp], acc.at[sidx.at[pp]],
                                  sg[pp]).wait()

        def compute_block(slot, pp):
            for t in range(BLK // 16):
                sl = pl.ds(t * 16, 16)
                sidx[pp, sl] = idxb[slot, 1, sl] + aoff
            @plsc.parallel_loop(0, BLK, 1, unroll=16)
            def _edge_loop(e):
                q0 = qb[slot, e, pl.ds(0, 16)]
                q1 = qb[slot, e, pl.ds(16, 16)]
                k0 = kvb[slot, e, pl.ds(0, 16)]
                k1 = kvb[slot, e, pl.ds(16, 16)]
                s_e = q0 * k0 + q1 * k1
                for r in (8, 4, 2, 1):
                    s_e = s_e + _rot(s_e, r)
                att = jnp.minimum(jnp.exp(s_e * SCALE), 5.0)
                msg[pp, e, pl.ds(0, 16)] = att
                msg[pp, e, pl.ds(8, 16)] = kvb[slot, e, pl.ds(32, 16)] * att
                msg[pp, e, pl.ds(24, 16)] = kvb[slot, e, pl.ds(48, 16)] * att
            pltpu.async_copy(msg.at[pp], acc.at[sidx.at[pp]], sg[pp],
                             add=True)

        idx_fire(0, 0)
        idx_fire(1, 1)
        idx_fire(2, 2)
        idx_wait(0, 0)
        offsets_and_gather(0)
        idx_wait(1, 1)
        offsets_and_gather(1)

        def _outer(o, carry2):
            for b in range(4):
                i = o * 4 + b

                @pl.when(i < NBLK - 3)
                def _():
                    idx_fire(i + 3, (b + 3) % 4)

                @pl.when(i < NBLK - 2)
                def _():
                    idx_wait(i + 2, (b + 2) % 4)
                    offsets_and_gather((b + 2) % 4)

                gather_wait(b)

                @pl.when(i >= 2)
                def _():
                    scatter_wait(b & 1)

                compute_block(b, b & 1)
            return carry2
        lax.fori_loop(0, NBLK // 4, _outer, 0)
        scatter_wait(0)
        scatter_wait(1)

        plsc.subcore_barrier()
        obase = head * NP + chunk * RPT
        pltpu.sync_copy(acc.at[pl.ds(base, RPT)],
                        out_hbm.at[pl.ds(obase, RPT)])
        return carry
    lax.fori_loop(0, 2, _pass, 0)


def _edge_phase(qh, kvh, ei_pad):
    mesh = plsc.VectorSubcoreMesh(core_axis_name="c", subcore_axis_name="s")
    kern = functools.partial(
        pl.kernel,
        out_type=jax.ShapeDtypeStruct((H * NP, ROWW), jnp.float32),
        mesh=mesh,
        compiler_params=pltpu.CompilerParams(use_tc_tiling_on_sc=False),
        scratch_types=[
            pltpu.VMEM((4, 4, BLK), jnp.int32),
            pltpu.VMEM((2, BLK), jnp.int32),
            pltpu.VMEM((4, BLK, DH), jnp.float32),
            pltpu.VMEM((4, BLK, 2 * DH), jnp.float32),
            pltpu.VMEM((2, BLK, ROWW), jnp.float32),
            pltpu.VMEM((BLK, ROWW), jnp.float32),
            pltpu.VMEM_SHARED((ACC_ROWS, ROWW), jnp.float32),
            pltpu.SemaphoreType.DMA,
            pltpu.SemaphoreType.DMA,
            pltpu.SemaphoreType.DMA,
            pltpu.SemaphoreType.DMA,
            pltpu.SemaphoreType.DMA,
            pltpu.SemaphoreType.DMA,
            pltpu.SemaphoreType.DMA,
            pltpu.SemaphoreType.DMA,
            pltpu.SemaphoreType.DMA,
            pltpu.SemaphoreType.DMA,
        ],
    )(_edge_body)
    return kern(qh, kvh, ei_pad)


_NB = 2000
_EPS = 1e-5


def _accum_stats(hm, s1_ref, s2_ref):
    colsum = jnp.sum(hm, axis=0, keepdims=True)
    colsq = jnp.sum(hm * hm, axis=0, keepdims=True)
    @pl.when(pl.program_id(0) == 0)
    def _():
        s1_ref[...] = colsum
        s2_ref[...] = colsq
    @pl.when(pl.program_id(0) != 0)
    def _():
        s1_ref[...] += colsum
        s2_ref[...] += colsq


def _t1_body(agg_ref, x_ref, wo_ref, bo_ref, h0_ref, s1_ref, s2_ref):
    parts = []
    for h in range(H):
        vagg = agg_ref[h, :, 8:8 + DH]
        parts.append(vagg / agg_ref[h, :, 0][:, None] + 1e-6)
    hm = jnp.concatenate(parts, axis=1)
    hm = jnp.dot(hm, wo_ref[...], preferred_element_type=jnp.float32)
    hm = hm + bo_ref[...] + x_ref[...]
    h0_ref[...] = hm
    _accum_stats(hm, s1_ref, s2_ref)


def _t2_body(h0_ref, s1_ref, s2_ref, g1_ref, be1_ref, w1_ref, b1_ref,
             w2_ref, b2_ref, h2_ref, t1_ref, t2_ref):
    mu = s1_ref[0] * (1.0 / N)
    var = s2_ref[0] * (1.0 / N) - mu * mu
    h1 = (h0_ref[...] - mu[None, :]) * jax.lax.rsqrt(var + _EPS)[None, :]
    h1 = h1 * g1_ref[...] + be1_ref[...]
    f = jnp.dot(h1, w1_ref[...], preferred_element_type=jnp.float32)
    f = jnp.maximum(f + b1_ref[...], 0.0)
    f = jnp.dot(f, w2_ref[...], preferred_element_type=jnp.float32)
    h2 = h1 + f + b2_ref[...]
    h2_ref[...] = h2
    _accum_stats(h2, t1_ref, t2_ref)


def _t3_body(h2_ref, t1_ref, t2_ref, g2_ref, be2_ref, o_ref):
    mu = t1_ref[0] * (1.0 / N)
    var = t2_ref[0] * (1.0 / N) - mu * mu
    h = (h2_ref[...] - mu[None, :]) * jax.lax.rsqrt(var + _EPS)[None, :]
    o_ref[...] = h * g2_ref[...] + be2_ref[...]


def _vec_spec(n):
    return pl.BlockSpec((1, n), lambda i: (0, 0))


def _row_spec(n):
    return pl.BlockSpec((_NB, n), lambda i: (i, 0))


def _full_spec(a, b):
    return pl.BlockSpec((a, b), lambda i: (0, 0))


def _tail(agg, x, wo, bo, w1, b1, w2, b2, g1, be1, g2, be2):
    grid = (N // _NB,)
    stats = jax.ShapeDtypeStruct((1, D_OUT), jnp.float32)
    h0, s1, s2 = pl.pallas_call(
        _t1_body,
        grid=grid,
        in_specs=[
            pl.BlockSpec((H, _NB, ROWW), lambda i: (0, i, 0)),
            _row_spec(D_IN),
            _full_spec(D_OUT, D_OUT),
            _vec_spec(D_OUT),
        ],
        out_specs=[_row_spec(D_OUT), _vec_spec(D_OUT), _vec_spec(D_OUT)],
        out_shape=[jax.ShapeDtypeStruct((N, D_OUT), jnp.float32), stats, stats],
    )(agg, x, wo, bo.reshape(1, D_OUT))
    h2, t1, t2 = pl.pallas_call(
        _t2_body,
        grid=grid,
        in_specs=[
            _row_spec(D_OUT), _vec_spec(D_OUT), _vec_spec(D_OUT),
            _vec_spec(D_OUT), _vec_spec(D_OUT),
            _full_spec(D_OUT, 2 * D_OUT), _vec_spec(2 * D_OUT),
            _full_spec(2 * D_OUT, D_OUT), _vec_spec(D_OUT),
        ],
        out_specs=[_row_spec(D_OUT), _vec_spec(D_OUT), _vec_spec(D_OUT)],
        out_shape=[jax.ShapeDtypeStruct((N, D_OUT), jnp.float32), stats, stats],
    )(h0, s1, s2, g1.reshape(1, -1), be1.reshape(1, -1), w1,
      b1.reshape(1, -1), w2, b2.reshape(1, -1))
    return pl.pallas_call(
        _t3_body,
        grid=grid,
        in_specs=[
            _row_spec(D_OUT), _vec_spec(D_OUT), _vec_spec(D_OUT),
            _vec_spec(D_OUT), _vec_spec(D_OUT),
        ],
        out_specs=_row_spec(D_OUT),
        out_shape=jax.ShapeDtypeStruct((N, D_OUT), jnp.float32),
    )(h2, t1, t2, g2.reshape(1, -1), be2.reshape(1, -1))


def kernel(node_feat, edge_index, Wq, Wk, Wv, Wo, bo, W1, b1, W2, b2,
           bn1_g, bn1_b, bn2_g, bn2_b):
    q, kv = _project(node_feat, Wq, Wk, Wv)
    qh = q.reshape(H * N, DH)
    kvh = kv.reshape(H * N, 2 * DH)
    ei_pad = jnp.concatenate(
        [edge_index, jnp.full((2, E_PAD - E), N, jnp.int32)], axis=1)
    agg = _edge_phase(qh, kvh, ei_pad).reshape(H, NP, ROWW)
    return _tail(agg, node_feat, Wo, bo, W1, b1, W2, b2,
                 bn1_g, bn1_b, bn2_g, bn2_b)


# speedup vs baseline: 1.1631x; 1.1631x over previous
import functools
import math

import jax
import jax.numpy as jnp
from jax import lax
from jax.experimental import pallas as pl
from jax.experimental.pallas import tpu as pltpu
from jax.experimental.pallas import tpu_sc as plsc

N = 10000
E = 160000
D_IN = 256
D_OUT = 256
H = 8
DH = 32
SCALE = 1.0 / math.sqrt(DH)

NP = 10016
BLK = 128
EC = 20480
NBLK = EC // BLK
E_PAD = 8 * EC
ROWW = 40
ACC_ROWS = 2 * NP
RPT = ACC_ROWS // 16


def _proj_body(x_ref, wq_ref, wk_ref, wv_ref, q_ref, kv_ref):
    xb = x_ref[...]
    q = jnp.dot(xb, wq_ref[...], preferred_element_type=jnp.float32)
    k = jnp.dot(xb, wk_ref[...], preferred_element_type=jnp.float32)
    v = jnp.dot(xb, wv_ref[...], preferred_element_type=jnp.float32)
    for h in range(H):
        q_ref[h] = q[:, h * DH:(h + 1) * DH]
        kv_ref[h, :, 0:DH] = k[:, h * DH:(h + 1) * DH]
        kv_ref[h, :, DH:2 * DH] = v[:, h * DH:(h + 1) * DH]


def _project(x, wq, wk, wv):
    nb = 2000
    grid = (N // nb,)
    return pl.pallas_call(
        _proj_body,
        grid=grid,
        in_specs=[
            pl.BlockSpec((nb, D_IN), lambda i: (i, 0)),
            pl.BlockSpec((D_IN, D_OUT), lambda i: (0, 0)),
            pl.BlockSpec((D_IN, D_OUT), lambda i: (0, 0)),
            pl.BlockSpec((D_IN, D_OUT), lambda i: (0, 0)),
        ],
        out_specs=[
            pl.BlockSpec((H, nb, DH), lambda i: (0, i, 0)),
            pl.BlockSpec((H, nb, 2 * DH), lambda i: (0, i, 0)),
        ],
        out_shape=[jax.ShapeDtypeStruct((H, N, DH), jnp.float32),
                   jax.ShapeDtypeStruct((H, N, 2 * DH), jnp.float32)],
    )(x, wq, wk, wv)


def _edge_body(q_hbm, kv_hbm, ei_hbm, out_hbm,
               idxb, sidx, qb, kvb, msg, zb, acc,
               ix0, ix1, ix2, ix3, g0, g1, g2, g3, s0, s1):
    ix = (ix0, ix1, ix2, ix3)
    g = (g0, g1, g2, g3)
    sg = (s0, s1)
    c = lax.axis_index("c")
    s = lax.axis_index("s")
    hloc = s // 8
    chunk = s % 8

    zeros16 = jnp.zeros((16,), jnp.float32)

    def _zrow(r, carry):
        for off in (0, 16, 24):
            zb[r, pl.ds(off, 16)] = zeros16
        return carry
    lax.fori_loop(0, BLK, _zrow, 0)

    base = s * RPT
    ebase = chunk * EC
    aoff = hloc * NP
    lanes = lax.iota(jnp.int32, 16)
    rot_idx = {r: (lanes + r) & 15 for r in (8, 4, 2, 1)}

    def _rot(a, r):
        return a.at[rot_idx[r]].get(mode="promise_in_bounds")

    def _pass(p, carry):
        head = c * 4 + p * 2 + hloc
        qoff = head * N

        nfull = RPT // BLK
        rem = RPT - nfull * BLK

        def _zacc(i, carry2):
            pltpu.async_copy(zb, acc.at[pl.ds(base + i * BLK, BLK)], ix[0])
            return carry2
        lax.fori_loop(0, nfull, _zacc, 0)
        if rem:
            pltpu.async_copy(zb.at[pl.ds(0, rem)],
                             acc.at[pl.ds(base + nfull * BLK, rem)], ix[0])

        def _zwait(i, carry2):
            pltpu.make_async_copy(
                zb, acc.at[pl.ds(base + i * BLK, BLK)], ix[0]).wait()
            return carry2
        lax.fori_loop(0, nfull, _zwait, 0)
        if rem:
            pltpu.make_async_copy(
                zb.at[pl.ds(0, rem)],
                acc.at[pl.ds(base + nfull * BLK, rem)], ix[0]).wait()
        plsc.subcore_barrier()

        def idx_fire(blk_i, slot):
            e0 = ebase + blk_i * BLK
            pltpu.async_copy(ei_hbm.at[0, pl.ds(e0, BLK)], idxb.at[slot, 0],
                             ix[slot])
            pltpu.async_copy(ei_hbm.at[1, pl.ds(e0, BLK)], idxb.at[slot, 1],
                             ix[slot])

        def idx_wait(blk_i, slot):
            e0 = ebase + blk_i * BLK
            pltpu.make_async_copy(ei_hbm.at[0, pl.ds(e0, BLK)],
                                  idxb.at[slot, 0], ix[slot]).wait()
            pltpu.make_async_copy(ei_hbm.at[1, pl.ds(e0, BLK)],
                                  idxb.at[slot, 1], ix[slot]).wait()

        def offsets_and_gather(slot):
            nmax = jnp.full((16,), N - 1, jnp.int32)
            for t in range(BLK // 16):
                sl = pl.ds(t * 16, 16)
                idxb[slot, 2, sl] = jnp.minimum(idxb[slot, 1, sl], nmax) + qoff
                idxb[slot, 3, sl] = jnp.minimum(idxb[slot, 0, sl], nmax) + qoff
            pltpu.async_copy(q_hbm.at[idxb.at[slot, 2]], qb.at[slot], g[slot])
            pltpu.async_copy(kv_hbm.at[idxb.at[slot, 3]], kvb.at[slot],
                             g[slot])

        def gather_wait(slot):
            pltpu.make_async_copy(q_hbm.at[idxb.at[slot, 2]], qb.at[slot],
                                  g[slot]).wait()
            pltpu.make_async_copy(kv_hbm.at[idxb.at[slot, 3]], kvb.at[slot],
                                  g[slot]).wait()

        def scatter_wait(pp):
            pltpu.make_async_copy(msg.at[pp], acc.at[sidx.at[pp]],
                                  sg[pp]).wait()

        def compute_block(slot, pp):
            for t in range(BLK // 16):
                sl = pl.ds(t * 16, 16)
                sidx[pp, sl] = idxb[slot, 1, sl] + aoff
            @plsc.parallel_loop(0, BLK, 1, unroll=8)
            def _edge_loop(e):
                q0 = qb[slot, e, pl.ds(0, 16)]
                q1 = qb[slot, e, pl.ds(16, 16)]
                k0 = kvb[slot, e, pl.ds(0, 16)]
                k1 = kvb[slot, e, pl.ds(16, 16)]
                s_e = q0 * k0 + q1 * k1
                for r in (8, 4, 2, 1):
                    s_e = s_e + _rot(s_e, r)
                att = jnp.minimum(jnp.exp(s_e * SCALE), 5.0)
                msg[pp, e, pl.ds(0, 16)] = att
                msg[pp, e, pl.ds(8, 16)] = kvb[slot, e, pl.ds(32, 16)] * att
                msg[pp, e, pl.ds(24, 16)] = kvb[slot, e, pl.ds(48, 16)] * att
            pltpu.async_copy(msg.at[pp], acc.at[sidx.at[pp]], sg[pp],
                             add=True)

        idx_fire(0, 0)
        idx_fire(1, 1)
        idx_fire(2, 2)
        idx_wait(0, 0)
        offsets_and_gather(0)
        idx_wait(1, 1)
        offsets_and_gather(1)

        def _outer(o, carry2):
            for b in range(4):
                i = o * 4 + b

                @pl.when(i < NBLK - 3)
                def _():
                    idx_fire(i + 3, (b + 3) % 4)

                @pl.when(i < NBLK - 2)
                def _():
                    idx_wait(i + 2, (b + 2) % 4)
                    offsets_and_gather((b + 2) % 4)

                gather_wait(b)

                @pl.when(i >= 2)
                def _():
                    scatter_wait(b & 1)

                compute_block(b, b & 1)
            return carry2
        lax.fori_loop(0, NBLK // 4, _outer, 0)
        scatter_wait(0)
        scatter_wait(1)

        plsc.subcore_barrier()
        obase = head * NP + chunk * RPT
        pltpu.sync_copy(acc.at[pl.ds(base, RPT)],
                        out_hbm.at[pl.ds(obase, RPT)])
        return carry
    lax.fori_loop(0, 2, _pass, 0)


def _edge_phase(qh, kvh, ei_pad):
    mesh = plsc.VectorSubcoreMesh(core_axis_name="c", subcore_axis_name="s")
    kern = functools.partial(
        pl.kernel,
        out_type=jax.ShapeDtypeStruct((H * NP, ROWW), jnp.float32),
        mesh=mesh,
        compiler_params=pltpu.CompilerParams(use_tc_tiling_on_sc=False),
        scratch_types=[
            pltpu.VMEM((4, 4, BLK), jnp.int32),
            pltpu.VMEM((2, BLK), jnp.int32),
            pltpu.VMEM((4, BLK, DH), jnp.float32),
            pltpu.VMEM((4, BLK, 2 * DH), jnp.float32),
            pltpu.VMEM((2, BLK, ROWW), jnp.float32),
            pltpu.VMEM((BLK, ROWW), jnp.float32),
            pltpu.VMEM_SHARED((ACC_ROWS, ROWW), jnp.float32),
            pltpu.SemaphoreType.DMA,
            pltpu.SemaphoreType.DMA,
            pltpu.SemaphoreType.DMA,
            pltpu.SemaphoreType.DMA,
            pltpu.SemaphoreType.DMA,
            pltpu.SemaphoreType.DMA,
            pltpu.SemaphoreType.DMA,
            pltpu.SemaphoreType.DMA,
            pltpu.SemaphoreType.DMA,
            pltpu.SemaphoreType.DMA,
        ],
    )(_edge_body)
    return kern(qh, kvh, ei_pad)


_NB = 2000
_EPS = 1e-5


def _accum_stats(hm, s1_ref, s2_ref):
    colsum = jnp.sum(hm, axis=0, keepdims=True)
    colsq = jnp.sum(hm * hm, axis=0, keepdims=True)
    @pl.when(pl.program_id(0) == 0)
    def _():
        s1_ref[...] = colsum
        s2_ref[...] = colsq
    @pl.when(pl.program_id(0) != 0)
    def _():
        s1_ref[...] += colsum
        s2_ref[...] += colsq


def _t1_body(agg_ref, x_ref, wo_ref, bo_ref, h0_ref, s1_ref, s2_ref):
    parts = []
    for h in range(H):
        vagg = agg_ref[h, :, 8:8 + DH]
        parts.append(vagg / agg_ref[h, :, 0][:, None] + 1e-6)
    hm = jnp.concatenate(parts, axis=1)
    hm = jnp.dot(hm, wo_ref[...], preferred_element_type=jnp.float32)
    hm = hm + bo_ref[...] + x_ref[...]
    h0_ref[...] = hm
    _accum_stats(hm, s1_ref, s2_ref)


def _t2_body(h0_ref, s1_ref, s2_ref, g1_ref, be1_ref, w1_ref, b1_ref,
             w2_ref, b2_ref, h2_ref, t1_ref, t2_ref):
    mu = s1_ref[0] * (1.0 / N)
    var = s2_ref[0] * (1.0 / N) - mu * mu
    h1 = (h0_ref[...] - mu[None, :]) * jax.lax.rsqrt(var + _EPS)[None, :]
    h1 = h1 * g1_ref[...] + be1_ref[...]
    f = jnp.dot(h1, w1_ref[...], preferred_element_type=jnp.float32)
    f = jnp.maximum(f + b1_ref[...], 0.0)
    f = jnp.dot(f, w2_ref[...], preferred_element_type=jnp.float32)
    h2 = h1 + f + b2_ref[...]
    h2_ref[...] = h2
    _accum_stats(h2, t1_ref, t2_ref)


def _t3_body(h2_ref, t1_ref, t2_ref, g2_ref, be2_ref, o_ref):
    mu = t1_ref[0] * (1.0 / N)
    var = t2_ref[0] * (1.0 / N) - mu * mu
    h = (h2_ref[...] - mu[None, :]) * jax.lax.rsqrt(var + _EPS)[None, :]
    o_ref[...] = h * g2_ref[...] + be2_ref[...]


def _vec_spec(n):
    return pl.BlockSpec((1, n), lambda i: (0, 0))


def _row_spec(n):
    return pl.BlockSpec((_NB, n), lambda i: (i, 0))


def _full_spec(a, b):
    return pl.BlockSpec((a, b), lambda i: (0, 0))


def _tail(agg, x, wo, bo, w1, b1, w2, b2, g1, be1, g2, be2):
    grid = (N // _NB,)
    stats = jax.ShapeDtypeStruct((1, D_OUT), jnp.float32)
    h0, s1, s2 = pl.pallas_call(
        _t1_body,
        grid=grid,
        in_specs=[
            pl.BlockSpec((H, _NB, ROWW), lambda i: (0, i, 0)),
            _row_spec(D_IN),
            _full_spec(D_OUT, D_OUT),
            _vec_spec(D_OUT),
        ],
        out_specs=[_row_spec(D_OUT), _vec_spec(D_OUT), _vec_spec(D_OUT)],
        out_shape=[jax.ShapeDtypeStruct((N, D_OUT), jnp.float32), stats, stats],
    )(agg, x, wo, bo.reshape(1, D_OUT))
    h2, t1, t2 = pl.pallas_call(
        _t2_body,
        grid=grid,
        in_specs=[
            _row_spec(D_OUT), _vec_spec(D_OUT), _vec_spec(D_OUT),
            _vec_spec(D_OUT), _vec_spec(D_OUT),
            _full_spec(D_OUT, 2 * D_OUT), _vec_spec(2 * D_OUT),
            _full_spec(2 * D_OUT, D_OUT), _vec_spec(D_OUT),
        ],
        out_specs=[_row_spec(D_OUT), _vec_spec(D_OUT), _vec_spec(D_OUT)],
        out_shape=[jax.ShapeDtypeStruct((N, D_OUT), jnp.float32), stats, stats],
    )(h0, s1, s2, g1.reshape(1, -1), be1.reshape(1, -1), w1,
      b1.reshape(1, -1), w2, b2.reshape(1, -1))
    return pl.pallas_call(
        _t3_body,
        grid=grid,
        in_specs=[
            _row_spec(D_OUT), _vec_spec(D_OUT), _vec_spec(D_OUT),
            _vec_spec(D_OUT), _vec_spec(D_OUT),
        ],
        out_specs=_row_spec(D_OUT),
        out_shape=jax.ShapeDtypeStruct((N, D_OUT), jnp.float32),
    )(h2, t1, t2, g2.reshape(1, -1), be2.reshape(1, -1))


def kernel(node_feat, edge_index, Wq, Wk, Wv, Wo, bo, W1, b1, W2, b2,
           bn1_g, bn1_b, bn2_g, bn2_b):
    q, kv = _project(node_feat, Wq, Wk, Wv)
    qh = q.reshape(H * N, DH)
    kvh = kv.reshape(H * N, 2 * DH)
    ei_pad = jnp.concatenate(
        [edge_index, jnp.full((2, E_PAD - E), N, jnp.int32)], axis=1)
    agg = _edge_phase(qh, kvh, ei_pad).reshape(H, NP, ROWW)
    return _tail(agg, node_feat, Wo, bo, W1, b1, W2, b2,
                 bn1_g, bn1_b, bn2_g, bn2_b)
